# baseline (speedup 1.0000x reference)
import dataclasses
import functools

import jax
import jax.numpy as jnp
from jax import lax
from jax.experimental import pallas as pl
from jax.experimental.pallas import tpu as pltpu
from jax.experimental.pallas import tpu_sc as plsc

N = 10000
E = 320000
DM = 128
DH = 64
H = 8
HC = 4
C = 16

NC = 2
NS = 16
L = 16

EPT = E // NS
EB = 80
NCHUNK = EPT // EB
RPT = 624
RTAIL = N - RPT * NS

ROWBLK = 400
NBLK = N // ROWBLK


def _hi_dot(a, b):
    return jnp.dot(a, b, precision=lax.Precision.HIGHEST,
                   preferred_element_type=jnp.float32)


def _tc_front_body(x_ref, wp_ref, bp_ref, w1_ref, ms_ref, md_ref,
                   h2_ref, ts_ref, td_ref):
    x = x_ref[...]
    xp = jnp.maximum(_hi_dot(x, wp_ref[...]) + bp_ref[...], 0.0)
    h = _hi_dot(xp, w1_ref[...])
    h2_ref[0, :, :] = h[:, :DH]
    h2_ref[1, :, :] = h[:, DH:]
    ts_ref[...] = _hi_dot(h, ms_ref[...])
    td_ref[...] = _hi_dot(h, md_ref[...])


def _tc_front(x, Wp, bp2, W1, Ms, Md):
    full = lambda shp: pl.BlockSpec(shp, lambda i: (0, 0))
    return pl.pallas_call(
        _tc_front_body,
        grid=(NBLK,),
        in_specs=[
            pl.BlockSpec((ROWBLK, DM), lambda i: (i, 0)),
            full((DM, DM)), full((1, DM)), full((DM, DM)),
            full((DM, 2 * H)), full((DM, 2 * H)),
        ],
        out_specs=[
            pl.BlockSpec((NC, ROWBLK, DH), lambda i: (0, i, 0)),
            pl.BlockSpec((ROWBLK, 2 * H), lambda i: (i, 0)),
            pl.BlockSpec((ROWBLK, 2 * H), lambda i: (i, 0)),
        ],
        out_shape=[
            jax.ShapeDtypeStruct((NC, N, DH), jnp.float32),
            jax.ShapeDtypeStruct((N, 2 * H), jnp.float32),
            jax.ShapeDtypeStruct((N, 2 * H), jnp.float32),
        ],
    )(x, Wp, bp2, W1, Ms, Md)


def _tc_combine_body(o2_ref, d_ref, h2_ref, ts_ref, td_ref,
                     b_ref, e8_ref, *rest):
    has_next = len(rest) > 1
    s = ts_ref[...] + td_ref[...]
    wself = jnp.exp(jnp.maximum(s, 0.2 * s))
    den8 = (d_ref[...] + wself)[:, :H]
    e8 = e8_ref[...]
    den128 = _hi_dot(den8, e8)
    wself128 = _hi_dot(wself[:, :H], e8)
    o128 = jnp.concatenate([o2_ref[0], o2_ref[1]], axis=1)
    h128 = jnp.concatenate([h2_ref[0], h2_ref[1]], axis=1)
    num = o128 + wself128 * h128
    o = jnp.maximum(num / den128 + b_ref[...], 0.0)
    if has_next:
        w2_ref, ms_ref, md_ref, h2o_ref, ts2_ref, td2_ref = rest
        h2 = _hi_dot(o, w2_ref[...])
        h2o_ref[0, :, :] = h2[:, :DH]
        h2o_ref[1, :, :] = h2[:, DH:]
        ts2_ref[...] = _hi_dot(h2, ms_ref[...])
        td2_ref[...] = _hi_dot(h2, md_ref[...])
    else:
        rest[0][...] = o


def _tc_combine(outp2, denp, h2, Ts, Td, b2d, E8, nxt=None):
    full = lambda shp: pl.BlockSpec(shp, lambda i: (0, 0))
    row64x2 = pl.BlockSpec((NC, ROWBLK, DH), lambda i: (0, i, 0))
    row128 = pl.BlockSpec((ROWBLK, DM), lambda i: (i, 0))
    row16 = pl.BlockSpec((ROWBLK, 2 * H), lambda i: (i, 0))
    in_specs = [row64x2, row16, row64x2, row16, row16,
                full((1, DM)), full((H, DM))]
    args = [outp2, denp, h2, Ts, Td, b2d, E8]
    if nxt is None:
        out_specs = [row128]
        out_shape = [jax.ShapeDtypeStruct((N, DM), jnp.float32)]
    else:
        W2, Ms2, Md2 = nxt
        in_specs += [full((DM, DM)), full((DM, 2 * H)), full((DM, 2 * H))]
        args += [W2, Ms2, Md2]
        out_specs = [row64x2, row16, row16]
        out_shape = [
            jax.ShapeDtypeStruct((NC, N, DH), jnp.float32),
            jax.ShapeDtypeStruct((N, 2 * H), jnp.float32),
            jax.ShapeDtypeStruct((N, 2 * H), jnp.float32),
        ]
    return pl.pallas_call(
        _tc_combine_body,
        grid=(NBLK,),
        in_specs=in_specs,
        out_specs=out_specs,
        out_shape=out_shape,
    )(*args)


def _sc_edge_body(h2_hbm, ts_hbm, td_hbm, src_hbm, dst_hbm, z64_hbm, z16_hbm,
                  outp_hbm, denp_hbm, *sc):
    src_i = list(sc[0:4])
    dst_i = list(sc[4:8])
    tsr = list(sc[8:10])
    tdr = list(sc[10:12])
    hr = list(sc[12:14])
    msg = list(sc[14:16])
    wbuf = list(sc[16:18])
    out_acc, den_acc = sc[18], sc[19]
    gsem = list(sc[20:22])
    ssem = list(sc[22:24])
    isem = list(sc[24:28])

    c = lax.axis_index("c")
    s = lax.axis_index("s")
    r0 = s * RPT
    c4 = c * HC

    pltpu.sync_copy(z64_hbm.at[pl.ds(r0, RPT)], out_acc.at[pl.ds(r0, RPT)])

    @pl.when(c == 0)
    def _zero_den():
        pltpu.sync_copy(z16_hbm.at[pl.ds(r0, RPT)], den_acc.at[pl.ds(r0, RPT)])

    @pl.when(s == NS - 1)
    def _zero_tail():
        rt = NS * RPT
        pltpu.sync_copy(z64_hbm.at[pl.ds(rt, RTAIL)],
                        out_acc.at[pl.ds(rt, RTAIL)])

        @pl.when(c == 0)
        def _zero_den_tail():
            pltpu.sync_copy(z16_hbm.at[pl.ds(rt, RTAIL)],
                            den_acc.at[pl.ds(rt, RTAIL)])

    plsc.subcore_barrier()

    ebase = s * EPT
    hview = h2_hbm.at[c]

    def i_issue(g, slot):
        base = ebase + g * EB
        pltpu.async_copy(src_hbm.at[pl.ds(base, EB)], src_i[slot], isem[slot])
        pltpu.async_copy(dst_hbm.at[pl.ds(base, EB)], dst_i[slot], isem[slot])

    def i_wait(g, slot):
        base = ebase + g * EB
        pltpu.make_async_copy(src_hbm.at[pl.ds(base, EB)], src_i[slot],
                              isem[slot]).wait()
        pltpu.make_async_copy(dst_hbm.at[pl.ds(base, EB)], dst_i[slot],
                              isem[slot]).wait()

    def g_issue(slot, d):
        pltpu.async_copy(ts_hbm.at[src_i[slot]], tsr[d], gsem[d])
        pltpu.async_copy(td_hbm.at[dst_i[slot]], tdr[d], gsem[d])
        pltpu.async_copy(hview.at[src_i[slot]], hr[d], gsem[d])

    def g_wait(slot, d):
        pltpu.make_async_copy(ts_hbm.at[src_i[slot]], tsr[d], gsem[d]).wait()
        pltpu.make_async_copy(td_hbm.at[dst_i[slot]], tdr[d], gsem[d]).wait()
        pltpu.make_async_copy(hview.at[src_i[slot]], hr[d], gsem[d]).wait()

    def s_issue(slot, d):
        pltpu.async_copy(msg[d], out_acc.at[dst_i[slot]], ssem[d], add=True)

        @pl.when(c == 0)
        def _den():
            pltpu.async_copy(wbuf[d], den_acc.at[dst_i[slot]], ssem[d],
                             add=True)

    def s_wait(slot, d):
        pltpu.make_async_copy(msg[d], out_acc.at[dst_i[slot]],
                              ssem[d]).wait()

        @pl.when(c == 0)
        def _den():
            pltpu.make_async_copy(wbuf[d], den_acc.at[dst_i[slot]],
                                  ssem[d]).wait()

    def compute(d):
        tsd, tdd, hd, msgd, wbd = tsr[d], tdr[d], hr[d], msg[d], wbuf[d]

        @plsc.parallel_loop(0, EB, unroll=8)
        def _edge(i):
            t = tsd[i] + tdd[i]
            w16 = jnp.exp(jnp.maximum(t, 0.2 * t))
            wbd[i] = w16
            for j in range(HC):
                jvec = jnp.full((L,), j, jnp.int32) + c4
                a = lax.gather(
                    w16, jvec[:, None],
                    lax.GatherDimensionNumbers(
                        offset_dims=(), collapsed_slice_dims=(0,),
                        start_index_map=(0,)),
                    slice_sizes=(1,),
                    mode=lax.GatherScatterMode.PROMISE_IN_BOUNDS)
                msgd[i, pl.ds(j * L, L)] = hd[i, pl.ds(j * L, L)] * a

    def step(g, m, do_swait, do_prefetch):
        d = m % 2
        nextslot = (m + 2) % 4
        if do_swait:
            s_wait(nextslot, d)
        if do_prefetch:
            i_issue(g + 2, nextslot)
        g_wait(m, d)
        compute(d)
        s_issue(m, d)
        if do_prefetch:
            i_wait(g + 2, nextslot)
            g_issue(nextslot, d)

    i_issue(0, 0)
    i_issue(1, 1)
    i_wait(0, 0)
    i_wait(1, 1)
    g_issue(0, 0)
    g_issue(1, 1)
    step(0, 0, False, True)
    step(1, 1, False, True)

    nmain = (NCHUNK - 6) // 4

    @pl.loop(0, nmain)
    def _main(k):
        g0 = 2 + 4 * k
        for j in range(4):
            step(g0 + j, (2 + j) % 4, True, True)

    for g in range(2 + 4 * nmain, NCHUNK):
        step(g, g % 4, True, g + 2 < NCHUNK)
    s_wait((NCHUNK - 2) % 4, (NCHUNK - 2) % 2)
    s_wait((NCHUNK - 1) % 4, (NCHUNK - 1) % 2)

    plsc.subcore_barrier()
    pltpu.sync_copy(out_acc.at[pl.ds(r0, RPT)], outp_hbm.at[c, pl.ds(r0, RPT)])

    @pl.when(c == 0)
    def _exp_den():
        pltpu.sync_copy(den_acc.at[pl.ds(r0, RPT)], denp_hbm.at[pl.ds(r0, RPT)])

    @pl.when(s == NS - 1)
    def _export_tail():
        rt = NS * RPT
        pltpu.sync_copy(out_acc.at[pl.ds(rt, RTAIL)],
                        outp_hbm.at[c, pl.ds(rt, RTAIL)])

        @pl.when(c == 0)
        def _exp_den_tail():
            pltpu.sync_copy(den_acc.at[pl.ds(rt, RTAIL)],
                            denp_hbm.at[pl.ds(rt, RTAIL)])


def _sc_compiler_params():
    cp = pltpu.CompilerParams()
    fields = pltpu.CompilerParams.__dataclass_fields__
    if "needs_layout_passes" in fields:
        cp = dataclasses.replace(cp, needs_layout_passes=False)
    if "use_tc_tiling_on_sc" in fields:
        cp = dataclasses.replace(cp, use_tc_tiling_on_sc=False)
    return cp


def _sc_edge(h2, Ts, Td, src, dst, z64, z16):
    mesh = plsc.VectorSubcoreMesh(core_axis_name="c", subcore_axis_name="s")
    kern = pl.kernel(
        _sc_edge_body,
        mesh=mesh,
        compiler_params=_sc_compiler_params(),
        out_type=[
            jax.ShapeDtypeStruct((NC, N, DH), jnp.float32),
            jax.ShapeDtypeStruct((N, 2 * H), jnp.float32),
        ],
        scratch_types=(
            [pltpu.VMEM((EB,), jnp.int32)] * 8
            + [pltpu.VMEM((EB, 2 * H), jnp.float32)] * 4
            + [pltpu.VMEM((EB, DH), jnp.float32)] * 4
            + [pltpu.VMEM((EB, 2 * H), jnp.float32)] * 2
            + [pltpu.VMEM_SHARED((N, DH), jnp.float32),
               pltpu.VMEM_SHARED((N, 2 * H), jnp.float32)]
            + [pltpu.SemaphoreType.DMA] * 8
        ),
    )
    return kern(h2, Ts, Td, src, dst, z64, z16)


def _expand_mat(a):
    eye8 = jnp.eye(H, dtype=jnp.float32)
    m = jnp.einsum('hc,hj->hcj', a, eye8).reshape(H * C, H)
    return jnp.concatenate([m, m], axis=1)


def kernel(x, edge_index, Wp, bp, W1, a1_src, a1_dst, b1,
           W2, a2_src, a2_dst, b2):
    src = edge_index[0]
    dst = edge_index[1]
    Ms1, Md1 = _expand_mat(a1_src), _expand_mat(a1_dst)
    Ms2, Md2 = _expand_mat(a2_src), _expand_mat(a2_dst)
    E8 = jnp.repeat(jnp.eye(H, dtype=jnp.float32), C, axis=1)
    z64 = jnp.zeros((N, DH), jnp.float32)
    z16 = jnp.zeros((N, 2 * H), jnp.float32)

    h1, Ts1, Td1 = _tc_front(x, Wp, bp[None, :], W1, Ms1, Md1)
    outp1, denp1 = _sc_edge(h1, Ts1, Td1, src, dst, z64, z16)
    h2, Ts2, Td2 = _tc_combine(outp1, denp1, h1, Ts1, Td1, b1[None, :],
                               E8, nxt=(W2, Ms2, Md2))
    outp2, denp2 = _sc_edge(h2, Ts2, Td2, src, dst, z64, z16)
    (o2,) = _tc_combine(outp2, denp2, h2, Ts2, Td2, b2[None, :], E8)
    return o2


# speedup vs baseline: 1.0035x; 1.0035x over previous
import dataclasses
import functools

import jax
import jax.numpy as jnp
from jax import lax
from jax.experimental import pallas as pl
from jax.experimental.pallas import tpu as pltpu
from jax.experimental.pallas import tpu_sc as plsc

N = 10000
E = 320000
DM = 128
DH = 64
H = 8
HC = 4
C = 16

NC = 2
NS = 16
L = 16

EPT = E // NS
EB = 80
NCHUNK = EPT // EB
RPT = 624
RTAIL = N - RPT * NS

ROWBLK = 400
NBLK = N // ROWBLK


def _hi_dot(a, b):
    return jnp.dot(a, b, precision=lax.Precision.HIGHEST,
                   preferred_element_type=jnp.float32)


def _tc_front_body(x_ref, wp_ref, bp_ref, w1_ref, ms_ref, md_ref,
                   h2_ref, ts_ref, td_ref):
    x = x_ref[...]
    xp = jnp.maximum(_hi_dot(x, wp_ref[...]) + bp_ref[...], 0.0)
    h = _hi_dot(xp, w1_ref[...])
    h2_ref[0, :, :] = h[:, :DH]
    h2_ref[1, :, :] = h[:, DH:]
    ts_ref[...] = _hi_dot(h, ms_ref[...])
    td_ref[...] = _hi_dot(h, md_ref[...])


def _tc_front(x, Wp, bp2, W1, Ms, Md):
    full = lambda shp: pl.BlockSpec(shp, lambda i: (0, 0))
    return pl.pallas_call(
        _tc_front_body,
        grid=(NBLK,),
        in_specs=[
            pl.BlockSpec((ROWBLK, DM), lambda i: (i, 0)),
            full((DM, DM)), full((1, DM)), full((DM, DM)),
            full((DM, 2 * H)), full((DM, 2 * H)),
        ],
        out_specs=[
            pl.BlockSpec((NC, ROWBLK, DH), lambda i: (0, i, 0)),
            pl.BlockSpec((ROWBLK, 2 * H), lambda i: (i, 0)),
            pl.BlockSpec((ROWBLK, 2 * H), lambda i: (i, 0)),
        ],
        out_shape=[
            jax.ShapeDtypeStruct((NC, N, DH), jnp.float32),
            jax.ShapeDtypeStruct((N, 2 * H), jnp.float32),
            jax.ShapeDtypeStruct((N, 2 * H), jnp.float32),
        ],
    )(x, Wp, bp2, W1, Ms, Md)


def _tc_combine_body(o2_ref, d_ref, h2_ref, ts_ref, td_ref,
                     b_ref, e8_ref, *rest):
    has_next = len(rest) > 1
    s = ts_ref[...] + td_ref[...]
    wself = jnp.exp(jnp.maximum(s, 0.2 * s))
    den8 = (d_ref[...] + wself)[:, :H]
    e8 = e8_ref[...]
    den128 = _hi_dot(den8, e8)
    wself128 = _hi_dot(wself[:, :H], e8)
    o128 = jnp.concatenate([o2_ref[0], o2_ref[1]], axis=1)
    h128 = jnp.concatenate([h2_ref[0], h2_ref[1]], axis=1)
    num = o128 + wself128 * h128
    o = jnp.maximum(num / den128 + b_ref[...], 0.0)
    if has_next:
        w2_ref, ms_ref, md_ref, h2o_ref, ts2_ref, td2_ref = rest
        h2 = _hi_dot(o, w2_ref[...])
        h2o_ref[0, :, :] = h2[:, :DH]
        h2o_ref[1, :, :] = h2[:, DH:]
        ts2_ref[...] = _hi_dot(h2, ms_ref[...])
        td2_ref[...] = _hi_dot(h2, md_ref[...])
    else:
        rest[0][...] = o


def _tc_combine(outp2, denp, h2, Ts, Td, b2d, E8, nxt=None):
    full = lambda shp: pl.BlockSpec(shp, lambda i: (0, 0))
    row64x2 = pl.BlockSpec((NC, ROWBLK, DH), lambda i: (0, i, 0))
    row128 = pl.BlockSpec((ROWBLK, DM), lambda i: (i, 0))
    row16 = pl.BlockSpec((ROWBLK, 2 * H), lambda i: (i, 0))
    in_specs = [row64x2, row16, row64x2, row16, row16,
                full((1, DM)), full((H, DM))]
    args = [outp2, denp, h2, Ts, Td, b2d, E8]
    if nxt is None:
        out_specs = [row128]
        out_shape = [jax.ShapeDtypeStruct((N, DM), jnp.float32)]
    else:
        W2, Ms2, Md2 = nxt
        in_specs += [full((DM, DM)), full((DM, 2 * H)), full((DM, 2 * H))]
        args += [W2, Ms2, Md2]
        out_specs = [row64x2, row16, row16]
        out_shape = [
            jax.ShapeDtypeStruct((NC, N, DH), jnp.float32),
            jax.ShapeDtypeStruct((N, 2 * H), jnp.float32),
            jax.ShapeDtypeStruct((N, 2 * H), jnp.float32),
        ]
    return pl.pallas_call(
        _tc_combine_body,
        grid=(NBLK,),
        in_specs=in_specs,
        out_specs=out_specs,
        out_shape=out_shape,
    )(*args)


def _sc_edge_body(h2_hbm, ts_hbm, td_hbm, src_hbm, dst_hbm, z64_hbm, z16_hbm,
                  outp_hbm, denp_hbm, *sc):
    src_i = list(sc[0:4])
    dst_i = list(sc[4:8])
    tsr = list(sc[8:10])
    tdr = list(sc[10:12])
    hr = list(sc[12:14])
    msg = list(sc[14:16])
    wbuf = list(sc[16:18])
    out_acc, den_acc = sc[18], sc[19]
    gsem = list(sc[20:22])
    ssem = list(sc[22:24])
    isem = list(sc[24:28])

    c = lax.axis_index("c")
    s = lax.axis_index("s")
    r0 = s * RPT
    c4 = c * HC

    pltpu.sync_copy(z64_hbm.at[pl.ds(r0, RPT)], out_acc.at[pl.ds(r0, RPT)])

    @pl.when(c == 0)
    def _zero_den():
        pltpu.sync_copy(z16_hbm.at[pl.ds(r0, RPT)], den_acc.at[pl.ds(r0, RPT)])

    @pl.when(s == NS - 1)
    def _zero_tail():
        rt = NS * RPT
        pltpu.sync_copy(z64_hbm.at[pl.ds(rt, RTAIL)],
                        out_acc.at[pl.ds(rt, RTAIL)])

        @pl.when(c == 0)
        def _zero_den_tail():
            pltpu.sync_copy(z16_hbm.at[pl.ds(rt, RTAIL)],
                            den_acc.at[pl.ds(rt, RTAIL)])

    plsc.subcore_barrier()

    ebase = s * EPT
    hview = h2_hbm.at[c]

    def i_issue(g, slot):
        base = ebase + g * EB
        pltpu.async_copy(src_hbm.at[pl.ds(base, EB)], src_i[slot], isem[slot])
        pltpu.async_copy(dst_hbm.at[pl.ds(base, EB)], dst_i[slot], isem[slot])

    def i_wait(g, slot):
        base = ebase + g * EB
        pltpu.make_async_copy(src_hbm.at[pl.ds(base, EB)], src_i[slot],
                              isem[slot]).wait()
        pltpu.make_async_copy(dst_hbm.at[pl.ds(base, EB)], dst_i[slot],
                              isem[slot]).wait()

    def g_issue(slot, d):
        pltpu.async_copy(ts_hbm.at[src_i[slot]], tsr[d], gsem[d])
        pltpu.async_copy(td_hbm.at[dst_i[slot]], tdr[d], gsem[d])
        pltpu.async_copy(hview.at[src_i[slot]], hr[d], gsem[d])

    def g_wait(slot, d):
        pltpu.make_async_copy(ts_hbm.at[src_i[slot]], tsr[d], gsem[d]).wait()
        pltpu.make_async_copy(td_hbm.at[dst_i[slot]], tdr[d], gsem[d]).wait()
        pltpu.make_async_copy(hview.at[src_i[slot]], hr[d], gsem[d]).wait()

    def s_issue(slot, d):
        pltpu.async_copy(msg[d], out_acc.at[dst_i[slot]], ssem[d], add=True)

        @pl.when(c == 0)
        def _den():
            pltpu.async_copy(wbuf[d], den_acc.at[dst_i[slot]], ssem[d],
                             add=True)

    def s_wait(slot, d):
        pltpu.make_async_copy(msg[d], out_acc.at[dst_i[slot]],
                              ssem[d]).wait()

        @pl.when(c == 0)
        def _den():
            pltpu.make_async_copy(wbuf[d], den_acc.at[dst_i[slot]],
                                  ssem[d]).wait()

    def compute(d):
        tsd, tdd, hd, msgd, wbd = tsr[d], tdr[d], hr[d], msg[d], wbuf[d]

        @plsc.parallel_loop(0, EB, unroll=4)
        def _edge(i):
            t = tsd[i] + tdd[i]
            w16 = jnp.exp(jnp.maximum(t, 0.2 * t))
            wbd[i] = w16
            for j in range(HC):
                jvec = jnp.full((L,), j, jnp.int32) + c4
                a = lax.gather(
                    w16, jvec[:, None],
                    lax.GatherDimensionNumbers(
                        offset_dims=(), collapsed_slice_dims=(0,),
                        start_index_map=(0,)),
                    slice_sizes=(1,),
                    mode=lax.GatherScatterMode.PROMISE_IN_BOUNDS)
                msgd[i, pl.ds(j * L, L)] = hd[i, pl.ds(j * L, L)] * a

    def step(g, m, do_swait, do_prefetch):
        d = m % 2
        nextslot = (m + 2) % 4
        if do_swait:
            s_wait(nextslot, d)
        if do_prefetch:
            i_issue(g + 2, nextslot)
        g_wait(m, d)
        compute(d)
        s_issue(m, d)
        if do_prefetch:
            i_wait(g + 2, nextslot)
            g_issue(nextslot, d)

    i_issue(0, 0)
    i_issue(1, 1)
    i_wait(0, 0)
    i_wait(1, 1)
    g_issue(0, 0)
    g_issue(1, 1)
    step(0, 0, False, True)
    step(1, 1, False, True)

    nmain = (NCHUNK - 6) // 4

    @pl.loop(0, nmain)
    def _main(k):
        g0 = 2 + 4 * k
        for j in range(4):
            step(g0 + j, (2 + j) % 4, True, True)

    for g in range(2 + 4 * nmain, NCHUNK):
        step(g, g % 4, True, g + 2 < NCHUNK)
    s_wait((NCHUNK - 2) % 4, (NCHUNK - 2) % 2)
    s_wait((NCHUNK - 1) % 4, (NCHUNK - 1) % 2)

    plsc.subcore_barrier()
    pltpu.sync_copy(out_acc.at[pl.ds(r0, RPT)], outp_hbm.at[c, pl.ds(r0, RPT)])

    @pl.when(c == 0)
    def _exp_den():
        pltpu.sync_copy(den_acc.at[pl.ds(r0, RPT)], denp_hbm.at[pl.ds(r0, RPT)])

    @pl.when(s == NS - 1)
    def _export_tail():
        rt = NS * RPT
        pltpu.sync_copy(out_acc.at[pl.ds(rt, RTAIL)],
                        outp_hbm.at[c, pl.ds(rt, RTAIL)])

        @pl.when(c == 0)
        def _exp_den_tail():
            pltpu.sync_copy(den_acc.at[pl.ds(rt, RTAIL)],
                            denp_hbm.at[pl.ds(rt, RTAIL)])


def _sc_compiler_params():
    cp = pltpu.CompilerParams()
    fields = pltpu.CompilerParams.__dataclass_fields__
    if "needs_layout_passes" in fields:
        cp = dataclasses.replace(cp, needs_layout_passes=False)
    if "use_tc_tiling_on_sc" in fields:
        cp = dataclasses.replace(cp, use_tc_tiling_on_sc=False)
    return cp


def _sc_edge(h2, Ts, Td, src, dst, z64, z16):
    mesh = plsc.VectorSubcoreMesh(core_axis_name="c", subcore_axis_name="s")
    kern = pl.kernel(
        _sc_edge_body,
        mesh=mesh,
        compiler_params=_sc_compiler_params(),
        out_type=[
            jax.ShapeDtypeStruct((NC, N, DH), jnp.float32),
            jax.ShapeDtypeStruct((N, 2 * H), jnp.float32),
        ],
        scratch_types=(
            [pltpu.VMEM((EB,), jnp.int32)] * 8
            + [pltpu.VMEM((EB, 2 * H), jnp.float32)] * 4
            + [pltpu.VMEM((EB, DH), jnp.float32)] * 4
            + [pltpu.VMEM((EB, 2 * H), jnp.float32)] * 2
            + [pltpu.VMEM_SHARED((N, DH), jnp.float32),
               pltpu.VMEM_SHARED((N, 2 * H), jnp.float32)]
            + [pltpu.SemaphoreType.DMA] * 8
        ),
    )
    return kern(h2, Ts, Td, src, dst, z64, z16)


def _expand_mat(a):
    eye8 = jnp.eye(H, dtype=jnp.float32)
    m = jnp.einsum('hc,hj->hcj', a, eye8).reshape(H * C, H)
    return jnp.concatenate([m, m], axis=1)


def kernel(x, edge_index, Wp, bp, W1, a1_src, a1_dst, b1,
           W2, a2_src, a2_dst, b2):
    src = edge_index[0]
    dst = edge_index[1]
    Ms1, Md1 = _expand_mat(a1_src), _expand_mat(a1_dst)
    Ms2, Md2 = _expand_mat(a2_src), _expand_mat(a2_dst)
    E8 = jnp.repeat(jnp.eye(H, dtype=jnp.float32), C, axis=1)
    z64 = jnp.zeros((N, DH), jnp.float32)
    z16 = jnp.zeros((N, 2 * H), jnp.float32)

    h1, Ts1, Td1 = _tc_front(x, Wp, bp[None, :], W1, Ms1, Md1)
    outp1, denp1 = _sc_edge(h1, Ts1, Td1, src, dst, z64, z16)
    h2, Ts2, Td2 = _tc_combine(outp1, denp1, h1, Ts1, Td1, b1[None, :],
                               E8, nxt=(W2, Ms2, Md2))
    outp2, denp2 = _sc_edge(h2, Ts2, Td2, src, dst, z64, z16)
    (o2,) = _tc_combine(outp2, denp2, h2, Ts2, Td2, b2[None, :], E8)
    return o2


# speedup vs baseline: 1.0204x; 1.0168x over previous
import dataclasses
import functools

import jax
import jax.numpy as jnp
from jax import lax
from jax.experimental import pallas as pl
from jax.experimental.pallas import tpu as pltpu
from jax.experimental.pallas import tpu_sc as plsc

N = 10000
E = 320000
DM = 128
DH = 64
DHX = 80
H = 8
HC = 4
C = 16

NC = 2
NS = 16
L = 16

EPT = E // NS
EB = 80
NCHUNK = EPT // EB
RPT = 624
RTAIL = N - RPT * NS

ROWBLK = 400
NBLK = N // ROWBLK


def _hi_dot(a, b):
    return jnp.dot(a, b, precision=lax.Precision.HIGHEST,
                   preferred_element_type=jnp.float32)


def _tc_front_body(x_ref, wp_ref, bp_ref, w1_ref, ms_ref, md_ref,
                   hx_ref, td_ref):
    x = x_ref[...]
    xp = jnp.maximum(_hi_dot(x, wp_ref[...]) + bp_ref[...], 0.0)
    h = _hi_dot(xp, w1_ref[...])
    ts = _hi_dot(h, ms_ref[...])
    hx_ref[0, :, :DH] = h[:, :DH]
    hx_ref[0, :, DH:] = ts
    hx_ref[1, :, :DH] = h[:, DH:]
    hx_ref[1, :, DH:] = ts
    td_ref[...] = _hi_dot(h, md_ref[...])


def _tc_front(x, Wp, bp2, W1, Ms, Md):
    full = lambda shp: pl.BlockSpec(shp, lambda i: (0, 0))
    return pl.pallas_call(
        _tc_front_body,
        grid=(NBLK,),
        in_specs=[
            pl.BlockSpec((ROWBLK, DM), lambda i: (i, 0)),
            full((DM, DM)), full((1, DM)), full((DM, DM)),
            full((DM, 2 * H)), full((DM, 2 * H)),
        ],
        out_specs=[
            pl.BlockSpec((NC, ROWBLK, DHX), lambda i: (0, i, 0)),
            pl.BlockSpec((ROWBLK, 2 * H), lambda i: (i, 0)),
        ],
        out_shape=[
            jax.ShapeDtypeStruct((NC, N, DHX), jnp.float32),
            jax.ShapeDtypeStruct((N, 2 * H), jnp.float32),
        ],
    )(x, Wp, bp2, W1, Ms, Md)


def _tc_combine_body(o2_ref, d_ref, hx_ref, td_ref,
                     b_ref, e8_ref, *rest):
    has_next = len(rest) > 1
    s = hx_ref[0, :, DH:] + td_ref[...]
    wself = jnp.exp(jnp.maximum(s, 0.2 * s))
    den8 = (d_ref[...] + wself)[:, :H]
    e8 = e8_ref[...]
    den128 = _hi_dot(den8, e8)
    wself128 = _hi_dot(wself[:, :H], e8)
    o128 = jnp.concatenate([o2_ref[0], o2_ref[1]], axis=1)
    h128 = jnp.concatenate([hx_ref[0, :, :DH], hx_ref[1, :, :DH]], axis=1)
    num = o128 + wself128 * h128
    o = jnp.maximum(num / den128 + b_ref[...], 0.0)
    if has_next:
        w2_ref, ms_ref, md_ref, hxo_ref, td2_ref = rest
        h2 = _hi_dot(o, w2_ref[...])
        ts2 = _hi_dot(h2, ms_ref[...])
        hxo_ref[0, :, :DH] = h2[:, :DH]
        hxo_ref[0, :, DH:] = ts2
        hxo_ref[1, :, :DH] = h2[:, DH:]
        hxo_ref[1, :, DH:] = ts2
        td2_ref[...] = _hi_dot(h2, md_ref[...])
    else:
        rest[0][...] = o


def _tc_combine(outp2, denp, hx, Td, b2d, E8, nxt=None):
    full = lambda shp: pl.BlockSpec(shp, lambda i: (0, 0))
    row64x2 = pl.BlockSpec((NC, ROWBLK, DH), lambda i: (0, i, 0))
    rowhx = pl.BlockSpec((NC, ROWBLK, DHX), lambda i: (0, i, 0))
    row128 = pl.BlockSpec((ROWBLK, DM), lambda i: (i, 0))
    row16 = pl.BlockSpec((ROWBLK, 2 * H), lambda i: (i, 0))
    in_specs = [row64x2, row16, rowhx, row16,
                full((1, DM)), full((H, DM))]
    args = [outp2, denp, hx, Td, b2d, E8]
    if nxt is None:
        out_specs = [row128]
        out_shape = [jax.ShapeDtypeStruct((N, DM), jnp.float32)]
    else:
        W2, Ms2, Md2 = nxt
        in_specs += [full((DM, DM)), full((DM, 2 * H)), full((DM, 2 * H))]
        args += [W2, Ms2, Md2]
        out_specs = [rowhx, row16]
        out_shape = [
            jax.ShapeDtypeStruct((NC, N, DHX), jnp.float32),
            jax.ShapeDtypeStruct((N, 2 * H), jnp.float32),
        ]
    return pl.pallas_call(
        _tc_combine_body,
        grid=(NBLK,),
        in_specs=in_specs,
        out_specs=out_specs,
        out_shape=out_shape,
    )(*args)


def _sc_edge_body(hx_hbm, td_hbm, ei_hbm, z64_hbm, z16_hbm,
                  outp_hbm, denp_hbm, *sc):
    idx2 = list(sc[0:4])
    tdr = list(sc[4:6])
    hr = list(sc[6:8])
    msg = list(sc[8:10])
    wbuf = list(sc[10:12])
    out_acc, den_acc = sc[12], sc[13]
    gsem = list(sc[14:16])
    ssem = list(sc[16:18])
    isem = list(sc[18:22])
    src_i = [r.at[0] for r in idx2]
    dst_i = [r.at[1] for r in idx2]

    c = lax.axis_index("c")
    s = lax.axis_index("s")
    r0 = s * RPT
    c4 = c * HC

    pltpu.sync_copy(z64_hbm.at[pl.ds(r0, RPT)], out_acc.at[pl.ds(r0, RPT)])

    @pl.when(c == 0)
    def _zero_den():
        pltpu.sync_copy(z16_hbm.at[pl.ds(r0, RPT)], den_acc.at[pl.ds(r0, RPT)])

    @pl.when(s == NS - 1)
    def _zero_tail():
        rt = NS * RPT
        pltpu.sync_copy(z64_hbm.at[pl.ds(rt, RTAIL)],
                        out_acc.at[pl.ds(rt, RTAIL)])

        @pl.when(c == 0)
        def _zero_den_tail():
            pltpu.sync_copy(z16_hbm.at[pl.ds(rt, RTAIL)],
                            den_acc.at[pl.ds(rt, RTAIL)])

    plsc.subcore_barrier()

    ebase = s * EPT
    hview = hx_hbm.at[c]

    def i_issue(g, slot):
        base = ebase + g * EB
        pltpu.async_copy(ei_hbm.at[:, pl.ds(base, EB)], idx2[slot],
                         isem[slot])

    def i_wait(g, slot):
        base = ebase + g * EB
        pltpu.make_async_copy(ei_hbm.at[:, pl.ds(base, EB)], idx2[slot],
                              isem[slot]).wait()

    def g_issue(slot, d):
        pltpu.async_copy(td_hbm.at[dst_i[slot]], tdr[d], gsem[d])
        pltpu.async_copy(hview.at[src_i[slot]], hr[d], gsem[d])

    def g_wait(slot, d):
        pltpu.make_async_copy(td_hbm.at[dst_i[slot]], tdr[d], gsem[d]).wait()
        pltpu.make_async_copy(hview.at[src_i[slot]], hr[d], gsem[d]).wait()

    def s_issue(slot, d):
        pltpu.async_copy(msg[d], out_acc.at[dst_i[slot]], ssem[d], add=True)

        @pl.when(c == 0)
        def _den():
            pltpu.async_copy(wbuf[d], den_acc.at[dst_i[slot]], ssem[d],
                             add=True)

    def s_wait(slot, d):
        pltpu.make_async_copy(msg[d], out_acc.at[dst_i[slot]],
                              ssem[d]).wait()

        @pl.when(c == 0)
        def _den():
            pltpu.make_async_copy(wbuf[d], den_acc.at[dst_i[slot]],
                                  ssem[d]).wait()

    def compute(d):
        tdd, hd, msgd, wbd = tdr[d], hr[d], msg[d], wbuf[d]

        @plsc.parallel_loop(0, EB, unroll=4)
        def _edge(i):
            t = hd[i, pl.ds(DH, L)] + tdd[i]
            w16 = jnp.exp(jnp.maximum(t, 0.2 * t))
            wbd[i] = w16
            for j in range(HC):
                jvec = jnp.full((L,), j, jnp.int32) + c4
                a = lax.gather(
                    w16, jvec[:, None],
                    lax.GatherDimensionNumbers(
                        offset_dims=(), collapsed_slice_dims=(0,),
                        start_index_map=(0,)),
                    slice_sizes=(1,),
                    mode=lax.GatherScatterMode.PROMISE_IN_BOUNDS)
                msgd[i, pl.ds(j * L, L)] = hd[i, pl.ds(j * L, L)] * a

    def step(g, m, do_swait, do_prefetch):
        d = m % 2
        nextslot = (m + 2) % 4
        if do_swait:
            s_wait(nextslot, d)
        if do_prefetch:
            i_issue(g + 2, nextslot)
        g_wait(m, d)
        compute(d)
        s_issue(m, d)
        if do_prefetch:
            i_wait(g + 2, nextslot)
            g_issue(nextslot, d)

    i_issue(0, 0)
    i_issue(1, 1)
    i_wait(0, 0)
    i_wait(1, 1)
    g_issue(0, 0)
    g_issue(1, 1)
    step(0, 0, False, True)
    step(1, 1, False, True)

    nmain = (NCHUNK - 6) // 4

    @pl.loop(0, nmain)
    def _main(k):
        g0 = 2 + 4 * k
        for j in range(4):
            step(g0 + j, (2 + j) % 4, True, True)

    for g in range(2 + 4 * nmain, NCHUNK):
        step(g, g % 4, True, g + 2 < NCHUNK)
    s_wait((NCHUNK - 2) % 4, (NCHUNK - 2) % 2)
    s_wait((NCHUNK - 1) % 4, (NCHUNK - 1) % 2)

    plsc.subcore_barrier()
    pltpu.sync_copy(out_acc.at[pl.ds(r0, RPT)], outp_hbm.at[c, pl.ds(r0, RPT)])

    @pl.when(c == 0)
    def _exp_den():
        pltpu.sync_copy(den_acc.at[pl.ds(r0, RPT)], denp_hbm.at[pl.ds(r0, RPT)])

    @pl.when(s == NS - 1)
    def _export_tail():
        rt = NS * RPT
        pltpu.sync_copy(out_acc.at[pl.ds(rt, RTAIL)],
                        outp_hbm.at[c, pl.ds(rt, RTAIL)])

        @pl.when(c == 0)
        def _exp_den_tail():
            pltpu.sync_copy(den_acc.at[pl.ds(rt, RTAIL)],
                            denp_hbm.at[pl.ds(rt, RTAIL)])


def _sc_compiler_params():
    cp = pltpu.CompilerParams()
    fields = pltpu.CompilerParams.__dataclass_fields__
    if "needs_layout_passes" in fields:
        cp = dataclasses.replace(cp, needs_layout_passes=False)
    if "use_tc_tiling_on_sc" in fields:
        cp = dataclasses.replace(cp, use_tc_tiling_on_sc=False)
    return cp


def _sc_edge(hx, Td, ei, z64, z16):
    mesh = plsc.VectorSubcoreMesh(core_axis_name="c", subcore_axis_name="s")
    kern = pl.kernel(
        _sc_edge_body,
        mesh=mesh,
        compiler_params=_sc_compiler_params(),
        out_type=[
            jax.ShapeDtypeStruct((NC, N, DH), jnp.float32),
            jax.ShapeDtypeStruct((N, 2 * H), jnp.float32),
        ],
        scratch_types=(
            [pltpu.VMEM((2, EB), jnp.int32)] * 4
            + [pltpu.VMEM((EB, 2 * H), jnp.float32)] * 2
            + [pltpu.VMEM((EB, DHX), jnp.float32)] * 2
            + [pltpu.VMEM((EB, DH), jnp.float32)] * 2
            + [pltpu.VMEM((EB, 2 * H), jnp.float32)] * 2
            + [pltpu.VMEM_SHARED((N, DH), jnp.float32),
               pltpu.VMEM_SHARED((N, 2 * H), jnp.float32)]
            + [pltpu.SemaphoreType.DMA] * 8
        ),
    )
    return kern(hx, Td, ei, z64, z16)


def _expand_mat(a):
    eye8 = jnp.eye(H, dtype=jnp.float32)
    m = jnp.einsum('hc,hj->hcj', a, eye8).reshape(H * C, H)
    return jnp.concatenate([m, m], axis=1)


def kernel(x, edge_index, Wp, bp, W1, a1_src, a1_dst, b1,
           W2, a2_src, a2_dst, b2):
    Ms1, Md1 = _expand_mat(a1_src), _expand_mat(a1_dst)
    Ms2, Md2 = _expand_mat(a2_src), _expand_mat(a2_dst)
    E8 = jnp.repeat(jnp.eye(H, dtype=jnp.float32), C, axis=1)
    z64 = jnp.zeros((N, DH), jnp.float32)
    z16 = jnp.zeros((N, 2 * H), jnp.float32)

    hx1, Td1 = _tc_front(x, Wp, bp[None, :], W1, Ms1, Md1)
    outp1, denp1 = _sc_edge(hx1, Td1, edge_index, z64, z16)
    hx2, Td2 = _tc_combine(outp1, denp1, hx1, Td1, b1[None, :],
                           E8, nxt=(W2, Ms2, Md2))
    outp2, denp2 = _sc_edge(hx2, Td2, edge_index, z64, z16)
    (o2,) = _tc_combine(outp2, denp2, hx2, Td2, b2[None, :], E8)
    return o2
